# baseline (speedup 1.0000x reference)
import functools

import jax
import jax.numpy as jnp
from jax.experimental import pallas as pl
from jax.experimental.pallas import tpu as pltpu

_IDX = 13
_NCHUNKS = 2


def kernel(inputs):
    batch, fields, dim = inputs.shape
    plane_rows = dim * batch // 128

    t = jnp.transpose(inputs, (1, 2, 0))
    t = t.reshape(fields, dim // 8, 8, batch // 128, 128)
    t = jnp.transpose(t, (0, 1, 3, 2, 4))
    rows = t.reshape(fields * plane_rows, 128)

    chunk = plane_rows // _NCHUNKS

    def _body(x_ref, o_ref, buf, in_sem, out_sem):
        for i in range(_NCHUNKS):
            pltpu.make_async_copy(
                x_ref.at[pl.ds(_IDX * plane_rows + i * chunk, chunk), :],
                buf.at[pl.ds(i * chunk, chunk), :],
                in_sem.at[i],
            ).start()
        for i in range(_NCHUNKS):
            pltpu.make_async_copy(
                x_ref.at[pl.ds(_IDX * plane_rows + i * chunk, chunk), :],
                buf.at[pl.ds(i * chunk, chunk), :],
                in_sem.at[i],
            ).wait()
            pltpu.make_async_copy(
                buf.at[pl.ds(i * chunk, chunk), :],
                o_ref.at[pl.ds(i * chunk, chunk), :],
                out_sem.at[i],
            ).start()
        for i in range(_NCHUNKS):
            pltpu.make_async_copy(
                buf.at[pl.ds(i * chunk, chunk), :],
                o_ref.at[pl.ds(i * chunk, chunk), :],
                out_sem.at[i],
            ).wait()

    out_rows = pl.pallas_call(
        _body,
        out_shape=jax.ShapeDtypeStruct((plane_rows, 128), inputs.dtype),
        in_specs=[pl.BlockSpec(memory_space=pl.MemorySpace.ANY)],
        out_specs=pl.BlockSpec(memory_space=pl.MemorySpace.ANY),
        scratch_shapes=[
            pltpu.VMEM((plane_rows, 128), inputs.dtype),
            pltpu.SemaphoreType.DMA((_NCHUNKS,)),
            pltpu.SemaphoreType.DMA((_NCHUNKS,)),
        ],
    )(rows)

    o = out_rows.reshape(dim // 8, batch // 128, 8, 128)
    o = jnp.transpose(o, (0, 2, 1, 3))
    o = o.reshape(dim, batch)
    return jnp.transpose(o, (1, 0))


# speedup vs baseline: 1.0485x; 1.0485x over previous
import functools

import jax
import jax.numpy as jnp
from jax.experimental import pallas as pl
from jax.experimental.pallas import tpu as pltpu

_IDX = 13
_NCHUNKS = 6


def kernel(inputs):
    batch, fields, dim = inputs.shape
    plane_rows = dim * batch // 128

    t = jnp.transpose(inputs, (1, 2, 0))
    t = t.reshape(fields, dim // 8, 8, batch // 128, 128)
    t = jnp.transpose(t, (0, 1, 3, 2, 4))
    rows = t.reshape(fields * plane_rows, 128)

    chunk = plane_rows // _NCHUNKS

    def _body(x_ref, o_ref, buf, in_sem, out_sem):
        for i in range(_NCHUNKS):
            pltpu.make_async_copy(
                x_ref.at[pl.ds(_IDX * plane_rows + i * chunk, chunk), :],
                buf.at[pl.ds(i * chunk, chunk), :],
                in_sem.at[i],
            ).start()
        for i in range(_NCHUNKS):
            pltpu.make_async_copy(
                x_ref.at[pl.ds(_IDX * plane_rows + i * chunk, chunk), :],
                buf.at[pl.ds(i * chunk, chunk), :],
                in_sem.at[i],
            ).wait()
            pltpu.make_async_copy(
                buf.at[pl.ds(i * chunk, chunk), :],
                o_ref.at[pl.ds(i * chunk, chunk), :],
                out_sem.at[i],
            ).start()
        for i in range(_NCHUNKS):
            pltpu.make_async_copy(
                buf.at[pl.ds(i * chunk, chunk), :],
                o_ref.at[pl.ds(i * chunk, chunk), :],
                out_sem.at[i],
            ).wait()

    out_rows = pl.pallas_call(
        _body,
        out_shape=jax.ShapeDtypeStruct((plane_rows, 128), inputs.dtype),
        in_specs=[pl.BlockSpec(memory_space=pl.MemorySpace.ANY)],
        out_specs=pl.BlockSpec(memory_space=pl.MemorySpace.ANY),
        scratch_shapes=[
            pltpu.VMEM((plane_rows, 128), inputs.dtype),
            pltpu.SemaphoreType.DMA((_NCHUNKS,)),
            pltpu.SemaphoreType.DMA((_NCHUNKS,)),
        ],
    )(rows)

    o = out_rows.reshape(dim // 8, batch // 128, 8, 128)
    o = jnp.transpose(o, (0, 2, 1, 3))
    o = o.reshape(dim, batch)
    return jnp.transpose(o, (1, 0))
